# baseline (speedup 1.0000x reference)
import jax
import jax.numpy as jnp
from jax.experimental import pallas as pl
from jax.experimental.pallas import tpu as pltpu

_N = 10000
_TM = 400
_NBLK = _N // _TM
_F32 = jnp.float32
_BF16 = jnp.bfloat16


def _lrelu(y):
    return jnp.where(y > 0, y, 0.01 * y)


def _sup1_body(x_ref, w_ref, o_ref):
    o_ref[...] = jnp.dot(
        x_ref[...], w_ref[...], preferred_element_type=_F32
    ).astype(_BF16)


def _layer1_body(a_ref, s_ref, w_ref, b_ref, a16_ref, sup_ref):
    a16 = a_ref[...].astype(_BF16)
    a16_ref[...] = a16
    y = jnp.dot(a16, s_ref[...], preferred_element_type=_F32) + b_ref[...]
    h = _lrelu(y)
    sup_ref[...] = jnp.dot(h, w_ref[...], preferred_element_type=_F32).astype(_BF16)


def _mega_body(a_ref, s2_ref, w3_ref, w4_ref, w5_ref, w6_ref, w7_ref, w8_ref,
               b2_ref, b3_ref, b4_ref, b5_ref, b6_ref, b7_ref, b8_ref,
               lat_ref, out_ref, supa_ref, supb_ref):
    l = pl.program_id(0)
    i = pl.program_id(1)
    a = a_ref[...]
    rows = pl.ds(i * _TM, _TM)

    def gc(s, b_ref):
        return jnp.dot(a, s, preferred_element_type=_F32) + b_ref[...]

    def store(ref, h, w_ref, d):
        ref[rows, :d] = jnp.dot(
            h, w_ref[...], preferred_element_type=_F32
        ).astype(_BF16)

    @pl.when(l == 0)
    def _():
        store(supa_ref, _lrelu(gc(s2_ref[...], b2_ref)), w3_ref, 16)

    @pl.when(l == 1)
    def _():
        store(supb_ref, _lrelu(gc(supa_ref[:, :16], b3_ref)), w4_ref, 8)

    @pl.when(l == 2)
    def _():
        y = gc(supb_ref[:, :8], b4_ref)
        lat_ref[rows, :] = y
        store(supa_ref, y, w5_ref, 16)

    @pl.when(l == 3)
    def _():
        store(supb_ref, _lrelu(gc(supa_ref[:, :16], b5_ref)), w6_ref, 32)

    @pl.when(l == 4)
    def _():
        store(supa_ref, _lrelu(gc(supb_ref[:, :32], b6_ref)), w7_ref, 64)

    @pl.when(l == 5)
    def _():
        store(supb_ref, _lrelu(gc(supa_ref[:, :64], b7_ref)), w8_ref, 128)

    @pl.when(l == 6)
    def _():
        out_ref[rows, :] = gc(supb_ref[:, :128], b8_ref)


def _row_spec(d):
    return pl.BlockSpec((_TM, d), lambda i: (i, 0))


def _const1(r, c):
    return pl.BlockSpec((r, c), lambda i: (0, 0))


def _const2(r, c):
    return pl.BlockSpec((r, c), lambda l, i: (0, 0))


def kernel(x, adj, inv_adj, W1, b1, W2, b2, W3, b3, W4, b4, W5, b5, W6, b6,
           W7, b7, W8, b8):
    del inv_adj
    n, d0 = x.shape
    bs = [b.reshape(1, -1) for b in (b1, b2, b3, b4, b5, b6, b7, b8)]

    sup1 = pl.pallas_call(
        _sup1_body,
        grid=(_NBLK,),
        in_specs=[_row_spec(d0), _const1(d0, 64)],
        out_specs=_row_spec(64),
        out_shape=jax.ShapeDtypeStruct((n, 64), _BF16),
    )(x, W1)

    adj16, sup2 = pl.pallas_call(
        _layer1_body,
        grid=(_NBLK,),
        in_specs=[_row_spec(n), _const1(n, 64), _const1(64, 32), _const1(1, 64)],
        out_specs=[_row_spec(n), _row_spec(32)],
        out_shape=[
            jax.ShapeDtypeStruct((n, n), _BF16),
            jax.ShapeDtypeStruct((n, 32), _BF16),
        ],
    )(adj, sup1, W2, bs[0])

    lat, out = pl.pallas_call(
        _mega_body,
        grid=(7, _NBLK),
        in_specs=[
            pl.BlockSpec((_TM, n), lambda l, i: (i, 0)),
            _const2(n, 32),
            _const2(32, 16), _const2(16, 8), _const2(8, 16), _const2(16, 32),
            _const2(32, 64), _const2(64, 128),
            _const2(1, 32), _const2(1, 16), _const2(1, 8), _const2(1, 16),
            _const2(1, 32), _const2(1, 64), _const2(1, 128),
        ],
        out_specs=[_const2(n, 8), _const2(n, 128)],
        out_shape=[
            jax.ShapeDtypeStruct((n, 8), _F32),
            jax.ShapeDtypeStruct((n, 128), _F32),
        ],
        scratch_shapes=[
            pltpu.VMEM((n, 64), _BF16),
            pltpu.VMEM((n, 128), _BF16),
        ],
    )(adj16, sup2, W3, W4, W5, W6, W7, W8, *bs[1:])

    return (lat, out)


# speedup vs baseline: 1.0638x; 1.0638x over previous
import jax
import jax.numpy as jnp
from jax.experimental import pallas as pl
from jax.experimental.pallas import tpu as pltpu

_N = 10000
_TM1 = 400
_TM = 1200
_F32 = jnp.float32
_BF16 = jnp.bfloat16
_PARAMS = pltpu.CompilerParams(vmem_limit_bytes=120 * 1024 * 1024)


def _lrelu(y):
    return jnp.where(y > 0, y, 0.01 * y)


def _sup1_body(x_ref, w_ref, o_ref):
    o_ref[...] = jnp.dot(
        x_ref[...], w_ref[...], preferred_element_type=_F32
    ).astype(_BF16)


def _layer1_body(a_ref, s_ref, w_ref, b_ref, a16_ref, sup_ref):
    a16 = a_ref[...].astype(_BF16)
    a16_ref[...] = a16
    y = jnp.dot(a16, s_ref[...], preferred_element_type=_F32) + b_ref[...]
    h = _lrelu(y)
    sup_ref[...] = jnp.dot(h, w_ref[...], preferred_element_type=_F32).astype(_BF16)


def _mid_body(a_ref, s_ref, w_ref, b_ref, sup_ref):
    y = jnp.dot(a_ref[...], s_ref[...], preferred_element_type=_F32) + b_ref[...]
    h = _lrelu(y)
    sup_ref[...] = jnp.dot(h, w_ref[...], preferred_element_type=_F32).astype(_BF16)


def _lat_body(a_ref, s_ref, w_ref, b_ref, lat_ref, sup_ref):
    y = jnp.dot(a_ref[...], s_ref[...], preferred_element_type=_F32) + b_ref[...]
    lat_ref[...] = y
    sup_ref[...] = jnp.dot(y, w_ref[...], preferred_element_type=_F32).astype(_BF16)


def _last_body(a_ref, s_ref, b_ref, out_ref):
    out_ref[...] = (
        jnp.dot(a_ref[...], s_ref[...], preferred_element_type=_F32) + b_ref[...]
    )


def _row_spec(tm, d):
    return pl.BlockSpec((tm, d), lambda i: (i, 0))


def _full_spec(r, c):
    return pl.BlockSpec((r, c), lambda i: (0, 0))


def kernel(x, adj, inv_adj, W1, b1, W2, b2, W3, b3, W4, b4, W5, b5, W6, b6,
           W7, b7, W8, b8):
    del inv_adj
    n, d0 = x.shape
    ws = [W1, W2, W3, W4, W5, W6, W7, W8]
    bs = [b.reshape(1, -1) for b in (b1, b2, b3, b4, b5, b6, b7, b8)]
    dims = [d0] + [w.shape[1] for w in ws]

    sup = pl.pallas_call(
        _sup1_body,
        grid=(pl.cdiv(n, _TM),),
        in_specs=[_row_spec(_TM, d0), _full_spec(d0, dims[1])],
        out_specs=_row_spec(_TM, dims[1]),
        out_shape=jax.ShapeDtypeStruct((n, dims[1]), _BF16),
        compiler_params=_PARAMS,
    )(x, W1)

    adj16, sup = pl.pallas_call(
        _layer1_body,
        grid=(n // _TM1,),
        in_specs=[
            _row_spec(_TM1, n),
            _full_spec(n, dims[1]),
            _full_spec(dims[1], dims[2]),
            _full_spec(1, dims[1]),
        ],
        out_specs=[_row_spec(_TM1, n), _row_spec(_TM1, dims[2])],
        out_shape=[
            jax.ShapeDtypeStruct((n, n), _BF16),
            jax.ShapeDtypeStruct((n, dims[2]), _BF16),
        ],
        compiler_params=_PARAMS,
    )(adj, sup, W2, bs[0])

    for li in (2, 3):
        sup = pl.pallas_call(
            _mid_body,
            grid=(pl.cdiv(n, _TM),),
            in_specs=[
                _row_spec(_TM, n),
                _full_spec(n, dims[li]),
                _full_spec(dims[li], dims[li + 1]),
                _full_spec(1, dims[li]),
            ],
            out_specs=_row_spec(_TM, dims[li + 1]),
            out_shape=jax.ShapeDtypeStruct((n, dims[li + 1]), _BF16),
            compiler_params=_PARAMS,
        )(adj16, sup, ws[li], bs[li - 1])

    lat, sup = pl.pallas_call(
        _lat_body,
        grid=(pl.cdiv(n, _TM),),
        in_specs=[
            _row_spec(_TM, n),
            _full_spec(n, dims[4]),
            _full_spec(dims[4], dims[5]),
            _full_spec(1, dims[4]),
        ],
        out_specs=[_row_spec(_TM, dims[4]), _row_spec(_TM, dims[5])],
        out_shape=[
            jax.ShapeDtypeStruct((n, dims[4]), _F32),
            jax.ShapeDtypeStruct((n, dims[5]), _BF16),
        ],
        compiler_params=_PARAMS,
    )(adj16, sup, W5, bs[3])

    for li in (5, 6, 7):
        sup = pl.pallas_call(
            _mid_body,
            grid=(pl.cdiv(n, _TM),),
            in_specs=[
                _row_spec(_TM, n),
                _full_spec(n, dims[li]),
                _full_spec(dims[li], dims[li + 1]),
                _full_spec(1, dims[li]),
            ],
            out_specs=_row_spec(_TM, dims[li + 1]),
            out_shape=jax.ShapeDtypeStruct((n, dims[li + 1]), _BF16),
            compiler_params=_PARAMS,
        )(adj16, sup, ws[li], bs[li - 1])

    out = pl.pallas_call(
        _last_body,
        grid=(pl.cdiv(n, _TM),),
        in_specs=[_row_spec(_TM, n), _full_spec(n, dims[8]), _full_spec(1, dims[8])],
        out_specs=_row_spec(_TM, dims[8]),
        out_shape=jax.ShapeDtypeStruct((n, dims[8]), _F32),
        compiler_params=_PARAMS,
    )(adj16, sup, bs[7])

    return (lat, out)
